# baseline (device time: 24709 ns/iter reference)
import jax
import jax.numpy as jnp
from jax import lax
from jax.experimental import pallas as pl
from jax.experimental.pallas import tpu as pltpu

N_DEV = 16


def kernel(x):
    m, n = x.shape
    rows = m // N_DEV

    def body(x_ref, out_ref, work, comm_ref, send_sems, recv_sems):
        my = lax.axis_index("i")

        barrier_sem = pltpu.get_barrier_semaphore()
        for o in range(1, N_DEV):
            pl.semaphore_signal(
                barrier_sem, inc=1,
                device_id=(lax.rem(my + o, N_DEV),),
                device_id_type=pl.DeviceIdType.MESH,
            )
        pl.semaphore_wait(barrier_sem, N_DEV - 1)

        work[...] = x_ref[...].astype(jnp.bfloat16)

        def peer_chunk(ref, p):
            return ref.at[pl.ds(p * rows, rows), :]

        far_first = (8, 7, 9, 6, 10, 5, 11, 4, 12, 3, 13, 2, 14, 1, 15)
        near_first = tuple(reversed(far_first))

        rs = {}
        for o in far_first:
            r = lax.rem(my - o + N_DEV, N_DEV)
            rdma = pltpu.make_async_remote_copy(
                src_ref=peer_chunk(work, r),
                dst_ref=comm_ref.at[o - 1],
                send_sem=send_sems.at[o - 1],
                recv_sem=recv_sems.at[o - 1],
                device_id=(r,),
                device_id_type=pl.DeviceIdType.MESH,
            )
            rdma.start()
            rs[o] = rdma
        acc = peer_chunk(work, my)[...]
        for o in near_first:
            rs[o].wait()
            acc = acc + comm_ref[o - 1]
        work[pl.ds(my * rows, rows), :] = acc

        ag = {}
        for o in far_first:
            r = lax.rem(my + o, N_DEV)
            rdma = pltpu.make_async_remote_copy(
                src_ref=peer_chunk(work, my),
                dst_ref=peer_chunk(work, my),
                send_sem=send_sems.at[N_DEV - 1 + o - 1],
                recv_sem=recv_sems.at[N_DEV - 1 + o - 1],
                device_id=(r,),
                device_id_type=pl.DeviceIdType.MESH,
            )
            rdma.start()
            ag[o] = rdma
        out_ref[pl.ds(my * rows, rows), :] = acc.astype(x_ref.dtype)
        for o in near_first:
            ag[o].wait()
            p = lax.rem(my - o + N_DEV, N_DEV)
            out_ref[pl.ds(p * rows, rows), :] = (
                peer_chunk(work, p)[...].astype(x_ref.dtype)
            )

    n_sems = 2 * (N_DEV - 1)
    return pl.pallas_call(
        body,
        out_shape=jax.ShapeDtypeStruct((m, n), x.dtype),
        in_specs=[pl.BlockSpec(memory_space=pltpu.VMEM)],
        out_specs=pl.BlockSpec(memory_space=pltpu.VMEM),
        scratch_shapes=[
            pltpu.VMEM((m, n), jnp.bfloat16),
            pltpu.VMEM((N_DEV - 1, rows, n), jnp.bfloat16),
            pltpu.SemaphoreType.DMA((n_sems,)),
            pltpu.SemaphoreType.DMA((n_sems,)),
        ],
        compiler_params=pltpu.CompilerParams(collective_id=0),
    )(x)


# device time: 24626 ns/iter; 1.0034x vs baseline; 1.0034x over previous
import jax
import jax.numpy as jnp
from jax import lax
from jax.experimental import pallas as pl
from jax.experimental.pallas import tpu as pltpu

N_DEV = 16


def kernel(x):
    m, n = x.shape
    rows = m // N_DEV

    def body(x_ref, out_ref, work, comm_ref, send_sems, recv_sems):
        my = lax.axis_index("i")

        barrier_sem = pltpu.get_barrier_semaphore()
        for o in range(1, N_DEV):
            pl.semaphore_signal(
                barrier_sem, inc=1,
                device_id=(lax.rem(my + o, N_DEV),),
                device_id_type=pl.DeviceIdType.MESH,
            )
        pl.semaphore_wait(barrier_sem, N_DEV - 1)

        work[...] = x_ref[...].astype(jnp.bfloat16)

        def peer_chunk(ref, p):
            return ref.at[pl.ds(p * rows, rows), :]

        far_first = (8, 7, 9, 6, 10, 5, 11, 4, 12, 3, 13, 2, 14, 1, 15)

        rs = []
        for o in far_first:
            r = lax.rem(my - o + N_DEV, N_DEV)
            rdma = pltpu.make_async_remote_copy(
                src_ref=peer_chunk(work, r),
                dst_ref=comm_ref.at[o - 1],
                send_sem=send_sems.at[o - 1],
                recv_sem=recv_sems.at[o - 1],
                device_id=(r,),
                device_id_type=pl.DeviceIdType.MESH,
            )
            rdma.start()
            rs.append(rdma)
        for rdma in rs:
            rdma.wait()
        acc = peer_chunk(work, my)[...]
        for o in range(1, N_DEV):
            acc = acc + comm_ref[o - 1]
        work[pl.ds(my * rows, rows), :] = acc

        ag = []
        for o in far_first:
            r = lax.rem(my + o, N_DEV)
            rdma = pltpu.make_async_remote_copy(
                src_ref=peer_chunk(work, my),
                dst_ref=peer_chunk(work, my),
                send_sem=send_sems.at[N_DEV - 1 + o - 1],
                recv_sem=recv_sems.at[N_DEV - 1 + o - 1],
                device_id=(r,),
                device_id_type=pl.DeviceIdType.MESH,
            )
            rdma.start()
            ag.append(rdma)
        for rdma in ag:
            rdma.wait()

        out_ref[...] = work[...].astype(x_ref.dtype)

    n_sems = 2 * (N_DEV - 1)
    return pl.pallas_call(
        body,
        out_shape=jax.ShapeDtypeStruct((m, n), x.dtype),
        in_specs=[pl.BlockSpec(memory_space=pltpu.VMEM)],
        out_specs=pl.BlockSpec(memory_space=pltpu.VMEM),
        scratch_shapes=[
            pltpu.VMEM((m, n), jnp.bfloat16),
            pltpu.VMEM((N_DEV - 1, rows, n), jnp.bfloat16),
            pltpu.SemaphoreType.DMA((n_sems,)),
            pltpu.SemaphoreType.DMA((n_sems,)),
        ],
        compiler_params=pltpu.CompilerParams(collective_id=0),
    )(x)


# device time: 20482 ns/iter; 1.2064x vs baseline; 1.2023x over previous
import jax
import jax.numpy as jnp
from jax import lax
from jax.experimental import pallas as pl
from jax.experimental.pallas import tpu as pltpu

N_DEV = 16


def kernel(x):
    m, n = x.shape
    rows = m // N_DEV

    def body(x_ref, out_ref, work, comm_ref, send_sems, recv_sems):
        my = lax.axis_index("i")

        barrier_sem = pltpu.get_barrier_semaphore()
        for o in range(1, N_DEV):
            pl.semaphore_signal(
                barrier_sem, inc=1,
                device_id=(lax.rem(my + o, N_DEV),),
                device_id_type=pl.DeviceIdType.MESH,
            )
        pl.semaphore_wait(barrier_sem, N_DEV - 1)

        work[...] = x_ref[...].astype(jnp.bfloat16)

        def peer_chunk(ref, p):
            return ref.at[pl.ds(p * rows, rows), :]

        rs = []
        for o in range(1, N_DEV):
            r = lax.rem(my - o + N_DEV, N_DEV)
            rdma = pltpu.make_async_remote_copy(
                src_ref=peer_chunk(work, r),
                dst_ref=comm_ref.at[o - 1],
                send_sem=send_sems.at[o - 1],
                recv_sem=recv_sems.at[o - 1],
                device_id=(r,),
                device_id_type=pl.DeviceIdType.MESH,
            )
            rdma.start()
            rs.append(rdma)
        for rdma in rs:
            rdma.wait()
        acc = peer_chunk(work, my)[...]
        for o in range(1, N_DEV):
            acc = acc + comm_ref[o - 1]
        work[pl.ds(my * rows, rows), :] = acc

        ag = []
        for o in range(1, N_DEV):
            r = lax.rem(my + o, N_DEV)
            rdma = pltpu.make_async_remote_copy(
                src_ref=peer_chunk(work, my),
                dst_ref=peer_chunk(work, my),
                send_sem=send_sems.at[N_DEV - 1 + o - 1],
                recv_sem=recv_sems.at[N_DEV - 1 + o - 1],
                device_id=(r,),
                device_id_type=pl.DeviceIdType.MESH,
            )
            rdma.start()
            ag.append(rdma)
        for rdma in ag:
            rdma.wait()

        out_ref[...] = work[...].astype(x_ref.dtype)

    n_sems = 2 * (N_DEV - 1)
    return pl.pallas_call(
        body,
        out_shape=jax.ShapeDtypeStruct((m, n), x.dtype),
        in_specs=[pl.BlockSpec(memory_space=pltpu.VMEM)],
        out_specs=pl.BlockSpec(memory_space=pltpu.VMEM),
        scratch_shapes=[
            pltpu.VMEM((m, n), jnp.bfloat16),
            pltpu.VMEM((N_DEV - 1, rows, n), jnp.bfloat16),
            pltpu.SemaphoreType.DMA((n_sems,)),
            pltpu.SemaphoreType.DMA((n_sems,)),
        ],
        compiler_params=pltpu.CompilerParams(collective_id=0),
    )(x)
